# baseline (speedup 1.0000x reference)
import functools

import jax
import jax.numpy as jnp
from jax import lax
from jax.experimental import pallas as pl
from jax.experimental.pallas import tpu as pltpu
from jax.experimental.pallas import tpu_sc as plsc

NC = 2
NS = 16
NW = NC * NS

B = 10000
BP = 10240
NPW = 320
STRIDE = 312
DEG = 32
FD = 128
ED = 64

GCH = 128
CH = GCH // DEG
NCH = NPW // CH
SCH = 64
NSCH = NPW // SCH

_mesh = plsc.VectorSubcoreMesh(core_axis_name="c", subcore_axis_name="s")


@functools.partial(
    pl.kernel,
    mesh=_mesh,
    out_type=[
        jax.ShapeDtypeStruct((BP, FD), jnp.float32),
        jax.ShapeDtypeStruct((BP, FD), jnp.float32),
    ],
    scratch_types=[
        pltpu.VMEM((NPW * DEG,), jnp.int32),
        pltpu.VMEM((NCH, GCH), jnp.int32),
        pltpu.VMEM((NPW,), jnp.int32),
        pltpu.VMEM((2, GCH, FD), jnp.float32),
        pltpu.VMEM((2, SCH, FD), jnp.float32),
        pltpu.VMEM_SHARED((NS * NPW, FD), jnp.float32),
        pltpu.SemaphoreType.DMA((2,)),
        pltpu.SemaphoreType.DMA((2,)),
        pltpu.SemaphoreType.DMA((4,)),
    ],
)
def _sc_agg(nidx_hbm, nodes_hbm, dest_hbm, zeros_hbm, feat_hbm,
            sum_hbm, self_hbm,
            idxs_v, didx_v, sidx_v, rows_v, srows_v, acc_v,
            gsem, ssem, psem):
    sid = lax.axis_index("s")
    wid = sid * NC + lax.axis_index("c")
    base = pl.multiple_of(
        lax.select(wid == NW - 1, jnp.int32(B - NPW), wid * STRIDE), 8)
    doff = pl.multiple_of(sid * NCH, NCH)
    abase = pl.multiple_of(sid * NPW, NPW)

    c1 = pltpu.async_copy(nidx_hbm.at[pl.ds(base * DEG, NPW * DEG)], idxs_v,
                          psem.at[0])
    c2 = pltpu.async_copy(dest_hbm.at[pl.ds(doff, NCH)], didx_v, psem.at[1])
    c3 = pltpu.async_copy(nodes_hbm.at[pl.ds(base, NPW)], sidx_v,
                          psem.at[2])
    c4 = pltpu.async_copy(zeros_hbm, acc_v.at[pl.ds(abase, NPW)], psem.at[3])
    c1.wait()
    pltpu.async_copy(feat_hbm.at[idxs_v.at[pl.ds(0, GCH)]], rows_v.at[0],
                     gsem.at[0])
    c2.wait()
    c3.wait()
    c4.wait()

    def pair(i, carry):
        for b in range(2):
            k = 2 * i + b
            pltpu.make_async_copy(
                feat_hbm.at[idxs_v.at[pl.ds(k * GCH, GCH)]], rows_v.at[b],
                gsem.at[b]).wait()

            @pl.when(k + 1 < NCH)
            def _():
                pltpu.async_copy(
                    feat_hbm.at[idxs_v.at[pl.ds((k + 1) * GCH, GCH)]],
                    rows_v.at[1 - b], gsem.at[1 - b])

            pltpu.sync_copy(rows_v.at[b], acc_v.at[didx_v.at[k]], add=True)
        return carry

    lax.fori_loop(0, NCH // 2, pair, 0)
    pltpu.sync_copy(acc_v.at[pl.ds(abase, NPW)], sum_hbm.at[pl.ds(base, NPW)])

    pltpu.async_copy(feat_hbm.at[sidx_v.at[pl.ds(0, SCH)]], srows_v.at[0],
                     ssem.at[0])
    for s in range(NSCH):
        b = s % 2
        pltpu.make_async_copy(
            feat_hbm.at[sidx_v.at[pl.ds(s * SCH, SCH)]], srows_v.at[b],
            ssem.at[b]).wait()
        if s + 1 < NSCH:
            pltpu.async_copy(
                feat_hbm.at[sidx_v.at[pl.ds((s + 1) * SCH, SCH)]],
                srows_v.at[1 - b], ssem.at[1 - b])
        pltpu.sync_copy(srows_v.at[b],
                        self_hbm.at[pl.ds(base + s * SCH, SCH)])


BLK = 1024


def _tc_body(sum_ref, self_ref, wi_ref, w1_ref, w2_ref, out_ref):
    agg = sum_ref[...] * (1.0 / DEG)
    r1 = jnp.maximum(
        jnp.dot(agg, wi_ref[...], preferred_element_type=jnp.float32), 0.0)
    comb = jnp.maximum(
        jnp.dot(self_ref[...], w1_ref[...], preferred_element_type=jnp.float32)
        + jnp.dot(r1, w2_ref[...], preferred_element_type=jnp.float32), 0.0)
    out_ref[...] = comb.T


_tc_combine = pl.pallas_call(
    _tc_body,
    grid=(BP // BLK,),
    in_specs=[
        pl.BlockSpec((BLK, FD), lambda i: (i, 0)),
        pl.BlockSpec((BLK, FD), lambda i: (i, 0)),
        pl.BlockSpec((FD, ED), lambda i: (0, 0)),
        pl.BlockSpec((FD, ED), lambda i: (0, 0)),
        pl.BlockSpec((ED, ED), lambda i: (0, 0)),
    ],
    out_specs=pl.BlockSpec((ED, BLK), lambda i: (0, i)),
    out_shape=jax.ShapeDtypeStruct((ED, BP), jnp.float32),
)


def kernel(nodes, labels, neigh_idx, features, W_intra, weight):
    nidx = jnp.reshape(neigh_idx, (-1,))
    dest = (jnp.repeat(jnp.arange(NPW, dtype=jnp.int32), DEG)
            .reshape(1, NCH, GCH)
            + (jnp.arange(NS, dtype=jnp.int32) * NPW)[:, None, None]
            ).reshape(NS * NCH, GCH)
    zeros = jnp.zeros((NPW, FD), jnp.float32)
    nsum, selff = _sc_agg(nidx, nodes, dest, zeros, features)
    out = _tc_combine(nsum, selff, W_intra, weight[:FD], weight[FD:])
    return out[:, :B]


# speedup vs baseline: 1.2262x; 1.2262x over previous
import functools

import jax
import jax.numpy as jnp
from jax import lax
from jax.experimental import pallas as pl
from jax.experimental.pallas import tpu as pltpu
from jax.experimental.pallas import tpu_sc as plsc

NC = 2
NS = 16
NW = NC * NS

B = 10000
BP = 10240
NPW = 320
STRIDE = 312
DEG = 32
FD = 128
ED = 64

GN = 64
NG = NPW // GN
NK = NG * DEG
NRING = 4
SCH = 64
NSCH = NPW // SCH

_mesh = plsc.VectorSubcoreMesh(core_axis_name="c", subcore_axis_name="s")


@functools.partial(
    pl.kernel,
    mesh=_mesh,
    out_type=[
        jax.ShapeDtypeStruct((BP, FD), jnp.float32),
        jax.ShapeDtypeStruct((BP, FD), jnp.float32),
    ],
    scratch_types=[
        pltpu.VMEM((NK, GN), jnp.int32),
        pltpu.VMEM((NPW,), jnp.int32),
        pltpu.VMEM((NPW, FD), jnp.float32),
        pltpu.VMEM((2, SCH, FD), jnp.float32),
        pltpu.SemaphoreType.DMA((NRING,)),
        pltpu.SemaphoreType.DMA((2,)),
        pltpu.SemaphoreType.DMA((2,)),
    ],
)
def _sc_agg(idxt_hbm, nodes_hbm, zeros_hbm, feat_hbm,
            sum_hbm, self_hbm,
            idxt_v, sidx_v, acc_v, srows_v, gsem, ssem, psem):
    sid = lax.axis_index("s")
    wid = sid * NC + lax.axis_index("c")
    base = pl.multiple_of(
        lax.select(wid == NW - 1, jnp.int32(B - NPW), wid * STRIDE), 8)
    toff = pl.multiple_of(wid * NK, NK)

    c1 = pltpu.async_copy(idxt_hbm.at[pl.ds(toff, NK)], idxt_v, psem.at[0])
    c2 = pltpu.async_copy(nodes_hbm.at[pl.ds(base, NPW)], sidx_v, psem.at[1])
    c3 = pltpu.async_copy(zeros_hbm, acc_v, gsem.at[NRING - 1])
    c1.wait()
    c3.wait()

    def gadd(t, b):
        g = lax.rem(t, NG)
        row = g * DEG + lax.div(t, NG)
        goff = pl.multiple_of(g * GN, GN)
        return pltpu.async_copy(
            feat_hbm.at[idxt_v.at[row]], acc_v.at[pl.ds(goff, GN)],
            gsem.at[b], add=True)

    for k in range(NRING - 1):
        gadd(jnp.int32(k), k)

    def gwait(t, b):
        g = lax.rem(t, NG)
        row = g * DEG + lax.div(t, NG)
        goff = pl.multiple_of(g * GN, GN)
        pltpu.make_async_copy(
            feat_hbm.at[idxt_v.at[row]], acc_v.at[pl.ds(goff, GN)],
            gsem.at[b]).wait()

    def step(t, carry):
        gadd(t + NRING - 1, lax.rem(t + NRING - 1, NRING))
        gwait(t, lax.rem(t, NRING))
        return carry

    lax.fori_loop(0, NK - (NRING - 1), step, 0)
    for t in range(NK - (NRING - 1), NK):
        gwait(jnp.int32(t), t % NRING)

    pltpu.sync_copy(acc_v, sum_hbm.at[pl.ds(base, NPW)])

    c2.wait()
    pltpu.async_copy(feat_hbm.at[sidx_v.at[pl.ds(0, SCH)]], srows_v.at[0],
                     ssem.at[0])
    for s in range(NSCH):
        b = s % 2
        pltpu.make_async_copy(
            feat_hbm.at[sidx_v.at[pl.ds(s * SCH, SCH)]], srows_v.at[b],
            ssem.at[b]).wait()
        if s + 1 < NSCH:
            pltpu.async_copy(
                feat_hbm.at[sidx_v.at[pl.ds((s + 1) * SCH, SCH)]],
                srows_v.at[1 - b], ssem.at[1 - b])
        pltpu.sync_copy(srows_v.at[b],
                        self_hbm.at[pl.ds(base + s * SCH, SCH)])


def _tidx_body(nidx_ref, out_ref):
    w = pl.program_id(0)
    base = lax.select(w == NW - 1, jnp.int32(B - NPW), w * STRIDE)
    for g in range(NG):
        sub = nidx_ref[pl.ds(base + g * GN, GN), :]
        out_ref[pl.ds(g * DEG, DEG), :] = sub.T


_tc_tidx = pl.pallas_call(
    _tidx_body,
    grid=(NW,),
    in_specs=[pl.BlockSpec((B, DEG), lambda w: (0, 0))],
    out_specs=pl.BlockSpec((NK, GN), lambda w: (w, 0)),
    out_shape=jax.ShapeDtypeStruct((NW * NK, GN), jnp.int32),
)


BLK = 1024


def _tc_body(sum_ref, self_ref, wi_ref, w1_ref, w2_ref, out_ref):
    agg = sum_ref[...] * (1.0 / DEG)
    r1 = jnp.maximum(
        jnp.dot(agg, wi_ref[...], preferred_element_type=jnp.float32), 0.0)
    comb = jnp.maximum(
        jnp.dot(self_ref[...], w1_ref[...], preferred_element_type=jnp.float32)
        + jnp.dot(r1, w2_ref[...], preferred_element_type=jnp.float32), 0.0)
    out_ref[...] = comb.T


_tc_combine = pl.pallas_call(
    _tc_body,
    grid=(BP // BLK,),
    in_specs=[
        pl.BlockSpec((BLK, FD), lambda i: (i, 0)),
        pl.BlockSpec((BLK, FD), lambda i: (i, 0)),
        pl.BlockSpec((FD, ED), lambda i: (0, 0)),
        pl.BlockSpec((FD, ED), lambda i: (0, 0)),
        pl.BlockSpec((ED, ED), lambda i: (0, 0)),
    ],
    out_specs=pl.BlockSpec((ED, BLK), lambda i: (0, i)),
    out_shape=jax.ShapeDtypeStruct((ED, BP), jnp.float32),
)


def kernel(nodes, labels, neigh_idx, features, W_intra, weight):
    zeros = jnp.zeros((NPW, FD), jnp.float32)
    idxt = _tc_tidx(neigh_idx)
    nsum, selff = _sc_agg(idxt, nodes, zeros, features)
    out = _tc_combine(nsum, selff, W_intra, weight[:FD], weight[FD:])
    return out[:, :B]
